# Initial kernel scaffold; baseline (speedup 1.0000x reference)
import jax
import jax.numpy as jnp
from jax.experimental import pallas as pl


def kernel(queries, image_feature_tensor, W1, b1, W2, b2, query_indices):
    raise NotImplementedError("write your pallas kernel here")


# speedup vs baseline: 7.4690x; 7.4690x over previous
import functools

import jax
import jax.numpy as jnp
from jax import lax
from jax.experimental import pallas as pl
from jax.experimental.pallas import tpu as pltpu
from jax.experimental.pallas import tpu_sc as plsc

_Q = 1024
_B = 4
_H = 128
_W = 128
_C = 256
_QD = 256
_HID = 512
_WIN = 5
_NPOS = _WIN * _WIN
_NC = 2
_NS = 16
_NWORK = _NC * _NS
_QPW = _Q // _NWORK
_LANES = 16
_NBUF = 4


def _sc_gather_body(b_hbm, y_hbm, x_hbm, feat_hbm, out_hbm,
                    b_v, y_v, x_v, idx_v, rows_v, gsem, ssem):
    wid = lax.axis_index("s") * _NC + lax.axis_index("c")
    qbase = wid * _QPW

    pltpu.sync_copy(b_hbm.at[pl.ds(qbase, _QPW)], b_v)
    pltpu.sync_copy(y_hbm.at[pl.ds(qbase, _QPW)], y_v)
    pltpu.sync_copy(x_hbm.at[pl.ds(qbase, _QPW)], x_v)

    for h in range(_QPW // _LANES):
        sl = pl.ds(h * _LANES, _LANES)
        bhi = b_v[sl] * (_H * _W)
        y = y_v[sl]
        x = x_v[sl]
        for j in range(_NPOS):
            dy = j // _WIN - _WIN // 2
            dx = j % _WIN - _WIN // 2
            yy = jnp.clip(y + dy, 0, _H - 1)
            xx = jnp.clip(x + dx, 0, _W - 1)
            idx_v[j, sl] = bhi + yy * _W + xx

    gathers = [None] * _NPOS
    stores = [None] * _NPOS
    for j in range(_NPOS):
        if j >= _NBUF:
            stores[j - _NBUF].wait()
        gathers[j] = pltpu.async_copy(
            feat_hbm.at[idx_v.at[j]], rows_v.at[j % _NBUF], gsem)
        if j >= 1:
            gathers[j - 1].wait()
            stores[j - 1] = pltpu.async_copy(
                rows_v.at[(j - 1) % _NBUF],
                out_hbm.at[j - 1, pl.ds(qbase, _QPW)], ssem)
    gathers[_NPOS - 1].wait()
    stores[_NPOS - 1] = pltpu.async_copy(
        rows_v.at[(_NPOS - 1) % _NBUF],
        out_hbm.at[_NPOS - 1, pl.ds(qbase, _QPW)], ssem)
    for j in range(_NPOS - _NBUF, _NPOS):
        stores[j].wait()


@jax.jit
def _sc_gather(bq, yq, xq, feat_flat):
    mesh = plsc.VectorSubcoreMesh(
        core_axis_name="c", subcore_axis_name="s",
        num_cores=_NC, num_subcores=_NS)
    return pl.kernel(
        _sc_gather_body,
        out_type=jax.ShapeDtypeStruct((_NPOS, _Q, _C), jnp.float32),
        mesh=mesh,
        scratch_types=[
            pltpu.VMEM((_QPW,), jnp.int32),
            pltpu.VMEM((_QPW,), jnp.int32),
            pltpu.VMEM((_QPW,), jnp.int32),
            pltpu.VMEM((_NPOS, _QPW), jnp.int32),
            pltpu.VMEM((_NBUF, _QPW, _C), jnp.float32),
            pltpu.SemaphoreType.DMA,
            pltpu.SemaphoreType.DMA,
        ],
    )(bq, yq, xq, feat_flat)


def _tc_body(yq_ref, xq_ref, keys_ref, w1_ref, b1_ref, q_ref, w2t_ref,
             b2_ref, out_ref, v_ref, qb2_ref):
    j = pl.program_id(0)

    @pl.when(j == 0)
    def _():
        v_ref[...] = jnp.dot(q_ref[...], w2t_ref[...],
                             preferred_element_type=jnp.float32)
        qb2_ref[...] = jnp.dot(q_ref[...], b2_ref[...],
                               preferred_element_type=jnp.float32)

    h = jnp.dot(keys_ref[0], w1_ref[...], preferred_element_type=jnp.float32)
    h = jnp.maximum(h + b1_ref[...], 0.0)
    col = jnp.sum(h * v_ref[...], axis=1, keepdims=True) + qb2_ref[...]

    dy = j // _WIN - _WIN // 2
    dx = j % _WIN - _WIN // 2
    yy = yq_ref[...] + dy
    xx = xq_ref[...] + dx
    inb = (yy >= 0) & (yy < _H) & (xx >= 0) & (xx < _W)
    out_ref[0] = jnp.where(inb, col, -jnp.inf)


@jax.jit
def _tc_compute(yq2, xq2, keys, W1, b1r, queries, W2t, b2c):
    return pl.pallas_call(
        _tc_body,
        grid=(_NPOS,),
        in_specs=[
            pl.BlockSpec((_Q, 1), lambda j: (0, 0)),
            pl.BlockSpec((_Q, 1), lambda j: (0, 0)),
            pl.BlockSpec((1, _Q, _C), lambda j: (j, 0, 0)),
            pl.BlockSpec((_C, _HID), lambda j: (0, 0)),
            pl.BlockSpec((1, _HID), lambda j: (0, 0)),
            pl.BlockSpec((_Q, _QD), lambda j: (0, 0)),
            pl.BlockSpec((_QD, _HID), lambda j: (0, 0)),
            pl.BlockSpec((_QD, 1), lambda j: (0, 0)),
        ],
        out_specs=pl.BlockSpec((1, _Q, 1), lambda j: (j, 0, 0)),
        out_shape=jax.ShapeDtypeStruct((_NPOS, _Q, 1), jnp.float32),
        scratch_shapes=[
            pltpu.VMEM((_Q, _HID), jnp.float32),
            pltpu.VMEM((_Q, 1), jnp.float32),
        ],
        compiler_params=pltpu.CompilerParams(
            dimension_semantics=("arbitrary",)),
    )(yq2, xq2, keys, W1, b1r, queries, W2t, b2c)


def kernel(queries, image_feature_tensor, W1, b1, W2, b2, query_indices):
    feat_flat = image_feature_tensor.reshape(_B * _H * _W, _C)
    bq = query_indices[:, 0]
    yq = query_indices[:, 1]
    xq = query_indices[:, 2]
    keys = _sc_gather(bq, yq, xq, feat_flat)
    out3 = _tc_compute(yq[:, None], xq[:, None], keys, W1,
                       b1.reshape(1, _HID), queries,
                       W2.T, b2.reshape(_QD, 1))
    return out3.reshape(_NPOS, _Q).T.reshape(_Q, _WIN, _WIN)
